# baseline (speedup 1.0000x reference)
import jax
import jax.numpy as jnp
from jax.experimental import pallas as pl

_C = 32000
_PAD = 0
_SM = 0.1
_CONF = 1.0 - _SM
_EPS = _SM / (_C - 2)
_BR = 64
_LW = 128
_NCH = _C // _LW


def _body(t_ref, x_ref, o_ref):
    tb = t_ref[...]
    lane = jax.lax.broadcasted_iota(jnp.int32, (_BR, _LW), 1)
    zero = jnp.zeros((_BR, _LW), jnp.float32)
    neg = jnp.full((_BR, _LW), -jnp.inf, jnp.float32)

    def pass_a(c, carry):
        m, sp, pt = carry
        x = x_ref[:, c, :]
        m = jnp.maximum(m, x)
        sp = sp + x
        pt = pt + jnp.where(c * _LW + lane == tb, x, 0.0)
        return (m, sp, pt)

    m128, sp128, pt128 = jax.lax.fori_loop(
        0, _NCH, pass_a, (neg, zero, zero), unroll=2)

    m = jnp.max(m128, axis=1, keepdims=True)

    def pass_b(c, s):
        x = x_ref[:, c, :]
        return s + jnp.exp(x - m)

    s128 = jax.lax.fori_loop(0, _NCH, pass_b, zero, unroll=2)

    z = m + jnp.log(jnp.sum(s128, axis=1, keepdims=True))
    sp = jnp.sum(sp128, axis=1, keepdims=True)
    pt = jnp.sum(pt128, axis=1, keepdims=True)
    x0 = x_ref[:, 0, :]
    p0 = jnp.sum(jnp.where(lane == 0, x0, 0.0), axis=1, keepdims=True)
    lt = pt - z
    l0 = p0 - z
    srow = sp - _C * z
    loss = -(_CONF * lt + _EPS * (srow - l0 - lt))
    o_ref[...] = jnp.where(tb == _PAD, 0.0, loss)


def kernel(pred, target):
    n = pred.shape[0]
    nb = n // _BR
    t2 = target.astype(jnp.int32).reshape(n, 1)
    x3 = pred.reshape(n, _NCH, _LW)
    rows = pl.pallas_call(
        _body,
        grid=(nb,),
        in_specs=[
            pl.BlockSpec((_BR, 1), lambda i: (i, 0)),
            pl.BlockSpec((_BR, _NCH, _LW), lambda i: (i, 0, 0)),
        ],
        out_specs=pl.BlockSpec((_BR, 1), lambda i: (i, 0)),
        out_shape=jax.ShapeDtypeStruct((n, 1), jnp.float32),
    )(t2, x3)
    return jnp.mean(rows)


# speedup vs baseline: 1.4004x; 1.4004x over previous
import jax
import jax.numpy as jnp
from jax.experimental import pallas as pl

_C = 32000
_PAD = 0
_SM = 0.1
_CONF = 1.0 - _SM
_EPS = _SM / (_C - 2)
_BR = 32
_LW = 128
_NCH = _C // _LW


def _body(t_ref, x_ref, o_ref):
    tb = t_ref[...]
    lane = jax.lax.broadcasted_iota(jnp.int32, (_BR, _LW), 1)

    m = x_ref[:, 0, :]
    sp = m
    pt = jnp.where(lane == tb, m, 0.0)
    for c in range(1, _NCH):
        x = x_ref[:, c, :]
        m = jnp.maximum(m, x)
        sp = sp + x
        pt = jnp.where(lane == tb - c * _LW, x, pt)

    mb = jnp.max(m, axis=1, keepdims=True)

    s = jnp.exp(x_ref[:, 0, :] - mb)
    for c in range(1, _NCH):
        s = s + jnp.exp(x_ref[:, c, :] - mb)

    z = mb + jnp.log(jnp.sum(s, axis=1, keepdims=True))
    spr = jnp.sum(sp, axis=1, keepdims=True)
    ptr = jnp.sum(pt, axis=1, keepdims=True)
    x0 = x_ref[:, 0, :]
    p0 = jnp.sum(jnp.where(lane == 0, x0, 0.0), axis=1, keepdims=True)
    lt = ptr - z
    l0 = p0 - z
    srow = spr - _C * z
    loss = -(_CONF * lt + _EPS * (srow - l0 - lt))
    o_ref[...] = jnp.where(tb == _PAD, 0.0, loss)


def kernel(pred, target):
    n = pred.shape[0]
    nb = n // _BR
    t2 = target.astype(jnp.int32).reshape(n, 1)
    x3 = pred.reshape(n, _NCH, _LW)
    rows = pl.pallas_call(
        _body,
        grid=(nb,),
        in_specs=[
            pl.BlockSpec((_BR, 1), lambda i: (i, 0)),
            pl.BlockSpec((_BR, _NCH, _LW), lambda i: (i, 0, 0)),
        ],
        out_specs=pl.BlockSpec((_BR, 1), lambda i: (i, 0)),
        out_shape=jax.ShapeDtypeStruct((n, 1), jnp.float32),
    )(t2, x3)
    return jnp.mean(rows)


# speedup vs baseline: 11.8468x; 8.4596x over previous
import jax
import jax.numpy as jnp
from jax.experimental import pallas as pl

_C = 32000
_PAD = 0
_SM = 0.1
_CONF = 1.0 - _SM
_EPS = _SM / (_C - 2)
_BR = 32
_LW = 128
_NCH = _C // _LW


def _body(t_ref, x_ref, o_ref):
    tb = t_ref[...]
    lane = jax.lax.broadcasted_iota(jnp.int32, (_BR, _LW), 1)

    m = x_ref[:, 0:_LW]
    sp = m
    pt = jnp.where(lane == tb, m, 0.0)
    for c in range(1, _NCH):
        x = x_ref[:, c * _LW:(c + 1) * _LW]
        m = jnp.maximum(m, x)
        sp = sp + x
        pt = jnp.where(lane == tb - c * _LW, x, pt)

    mb = jnp.max(m, axis=1, keepdims=True)

    s = jnp.exp(x_ref[:, 0:_LW] - mb)
    for c in range(1, _NCH):
        s = s + jnp.exp(x_ref[:, c * _LW:(c + 1) * _LW] - mb)

    z = mb + jnp.log(jnp.sum(s, axis=1, keepdims=True))
    spr = jnp.sum(sp, axis=1, keepdims=True)
    ptr = jnp.sum(pt, axis=1, keepdims=True)
    x0 = x_ref[:, 0:_LW]
    p0 = jnp.sum(jnp.where(lane == 0, x0, 0.0), axis=1, keepdims=True)
    lt = ptr - z
    l0 = p0 - z
    srow = spr - _C * z
    loss = -(_CONF * lt + _EPS * (srow - l0 - lt))
    o_ref[...] = jnp.where(tb == _PAD, 0.0, loss)


def kernel(pred, target):
    n = pred.shape[0]
    nb = n // _BR
    t2 = target.astype(jnp.int32).reshape(n, 1)
    rows = pl.pallas_call(
        _body,
        grid=(nb,),
        in_specs=[
            pl.BlockSpec((_BR, 1), lambda i: (i, 0)),
            pl.BlockSpec((_BR, _C), lambda i: (i, 0)),
        ],
        out_specs=pl.BlockSpec((_BR, 1), lambda i: (i, 0)),
        out_shape=jax.ShapeDtypeStruct((n, 1), jnp.float32),
    )(t2, pred)
    return jnp.mean(rows)
